# baseline (speedup 1.0000x reference)
import functools

import jax
import jax.numpy as jnp
import numpy as np
from jax.experimental import pallas as pl

B = 4
N_POINT = 4096
N_CENTER = 1024
N_NEAR = 32
C_IN = 128
C_MID = 183
C_OUT = 256

BM = 512
M_TOTAL = B * N_CENTER * N_NEAR
N_ROWS_F = float(M_TOTAL)
EPS = 1e-5


def _knn_jax(xyz, k):
    sq = jnp.sum(xyz * xyz, axis=-1)
    dist = sq[:, :, None] + sq[:, None, :] - 2.0 * jnp.einsum('bnc,bmc->bnm', xyz, xyz)
    _, idx = jax.lax.top_k(-dist, k)
    return idx


def _fps_jax(xyz, n_center):
    b, n, _ = xyz.shape

    def step(carry, _):
        dist, far = carry
        centroid = jnp.take_along_axis(xyz, far[:, None, None].astype(jnp.int32), axis=1)
        d = jnp.sum((xyz - centroid) ** 2, axis=-1)
        dist = jnp.minimum(dist, d)
        new_far = jnp.argmax(dist, axis=-1).astype(jnp.int32)
        return (dist, new_far), far

    init = (jnp.full((b, n), 1e10, dtype=jnp.float32), jnp.zeros((b,), dtype=jnp.int32))
    _, idxs = jax.lax.scan(step, init, None, length=n_center)
    return jnp.transpose(idxs)


def _index_points(points, idx):
    return jax.vmap(lambda p, i: p[i])(points, idx)


def _mlp1_body(gfea_ref, gxyz_ref, w1a_ref, w1b_ref, b1_ref,
               h1_ref, stats_ref):
    i = pl.program_id(0)
    h = (jnp.dot(gfea_ref[...], w1a_ref[...], preferred_element_type=jnp.float32)
         + jnp.dot(gxyz_ref[...], w1b_ref[...], preferred_element_type=jnp.float32)
         + b1_ref[...])
    h1_ref[...] = h
    s = jnp.sum(h, axis=0, keepdims=True)
    ss = jnp.sum(h * h, axis=0, keepdims=True)
    upd = jnp.concatenate([s, ss], axis=0)

    @pl.when(i == 0)
    def _():
        stats_ref[...] = upd

    @pl.when(i > 0)
    def _():
        stats_ref[...] += upd


def _mlp_pass1(gfea, gxyz, W1a, W1b, b1):
    grid = (M_TOTAL // BM,)
    return pl.pallas_call(
        _mlp1_body,
        grid=grid,
        in_specs=[
            pl.BlockSpec((BM, C_IN), lambda i: (i, 0)),
            pl.BlockSpec((BM, 8), lambda i: (i, 0)),
            pl.BlockSpec((C_IN, C_MID), lambda i: (0, 0)),
            pl.BlockSpec((8, C_MID), lambda i: (0, 0)),
            pl.BlockSpec((1, C_MID), lambda i: (0, 0)),
        ],
        out_specs=[
            pl.BlockSpec((BM, C_MID), lambda i: (i, 0)),
            pl.BlockSpec((2, C_MID), lambda i: (0, 0)),
        ],
        out_shape=[
            jax.ShapeDtypeStruct((M_TOTAL, C_MID), jnp.float32),
            jax.ShapeDtypeStruct((2, C_MID), jnp.float32),
        ],
    )(gfea, gxyz, W1a, W1b, b1)


def _mlp2_body(h1_ref, stats1_ref, w2_ref, b2_ref, g1_ref, be1_ref,
               maxh2_ref, stats2_ref):
    i = pl.program_id(0)
    m1 = stats1_ref[0:1, :] / N_ROWS_F
    var1 = stats1_ref[1:2, :] / N_ROWS_F - m1 * m1
    inv1 = g1_ref[...] * jax.lax.rsqrt(var1 + EPS)
    a = jnp.maximum((h1_ref[...] - m1) * inv1 + be1_ref[...], 0.0)
    h2 = jnp.dot(a, w2_ref[...], preferred_element_type=jnp.float32) + b2_ref[...]
    s = jnp.sum(h2, axis=0, keepdims=True)
    ss = jnp.sum(h2 * h2, axis=0, keepdims=True)
    upd = jnp.concatenate([s, ss], axis=0)
    maxh2_ref[...] = jnp.max(h2.reshape(BM // N_NEAR, N_NEAR, C_OUT), axis=1)

    @pl.when(i == 0)
    def _():
        stats2_ref[...] = upd

    @pl.when(i > 0)
    def _():
        stats2_ref[...] += upd


def _mlp_pass2(h1, stats1, W2, b2, g1, be1):
    grid = (M_TOTAL // BM,)
    return pl.pallas_call(
        _mlp2_body,
        grid=grid,
        in_specs=[
            pl.BlockSpec((BM, C_MID), lambda i: (i, 0)),
            pl.BlockSpec((2, C_MID), lambda i: (0, 0)),
            pl.BlockSpec((C_MID, C_OUT), lambda i: (0, 0)),
            pl.BlockSpec((1, C_OUT), lambda i: (0, 0)),
            pl.BlockSpec((1, C_MID), lambda i: (0, 0)),
            pl.BlockSpec((1, C_MID), lambda i: (0, 0)),
        ],
        out_specs=[
            pl.BlockSpec((BM // N_NEAR, C_OUT), lambda i: (i, 0)),
            pl.BlockSpec((2, C_OUT), lambda i: (0, 0)),
        ],
        out_shape=[
            jax.ShapeDtypeStruct((B * N_CENTER, C_OUT), jnp.float32),
            jax.ShapeDtypeStruct((2, C_OUT), jnp.float32),
        ],
    )(h1, stats1, W2, b2, g1, be1)


def _epi_body(x_ref, stats2_ref, g2_ref, be2_ref, o_ref):
    m2 = stats2_ref[0:1, :] / N_ROWS_F
    var2 = stats2_ref[1:2, :] / N_ROWS_F - m2 * m2
    inv2 = g2_ref[...] * jax.lax.rsqrt(var2 + EPS)
    o_ref[...] = jnp.maximum((x_ref[...] - m2) * inv2 + be2_ref[...], 0.0)


def _mlp_epilogue(maxh2, stats2, g2, be2):
    grid = (8,)
    bm = (B * N_CENTER) // 8
    return pl.pallas_call(
        _epi_body,
        grid=grid,
        in_specs=[
            pl.BlockSpec((bm, C_OUT), lambda i: (i, 0)),
            pl.BlockSpec((2, C_OUT), lambda i: (0, 0)),
            pl.BlockSpec((1, C_OUT), lambda i: (0, 0)),
            pl.BlockSpec((1, C_OUT), lambda i: (0, 0)),
        ],
        out_specs=pl.BlockSpec((bm, C_OUT), lambda i: (i, 0)),
        out_shape=jax.ShapeDtypeStruct((B * N_CENTER, C_OUT), jnp.float32),
    )(maxh2, stats2, g2, be2)


def kernel(xyz, fea, W1, b1, g1, be1, W2, b2, g2, be2):
    idx_all = _knn_jax(xyz, N_NEAR)
    fps_idx = _fps_jax(xyz, N_CENTER)
    center_xyz = _index_points(xyz, fps_idx)
    group_idx = _index_points(idx_all, fps_idx)
    group_xyz = _index_points(xyz, group_idx)
    xyz_rel = group_xyz - center_xyz[:, :, None, :]
    group_fea = _index_points(fea, group_idx)

    gfea = group_fea.reshape(M_TOTAL, C_IN)
    gxyz = jnp.pad(xyz_rel.reshape(M_TOTAL, 3), ((0, 0), (0, 5)))

    W1a = W1[:C_IN]
    W1b = jnp.pad(W1[C_IN:], ((0, 5), (0, 0)))
    h1, stats1 = _mlp_pass1(gfea, gxyz, W1a, W1b, b1[None, :])
    maxh2, stats2 = _mlp_pass2(h1, stats1, W2, b2[None, :], g1[None, :], be1[None, :])
    out = _mlp_epilogue(maxh2, stats2, g2[None, :], be2[None, :])
    return (center_xyz, out.reshape(B, N_CENTER, C_OUT))


# speedup vs baseline: 2.9883x; 2.9883x over previous
import functools

import jax
import jax.numpy as jnp
import numpy as np
from jax.experimental import pallas as pl

B = 4
N_POINT = 4096
N_CENTER = 1024
N_NEAR = 32
C_IN = 128
C_MID = 183
C_OUT = 256

BM = 512
M_TOTAL = B * N_CENTER * N_NEAR
N_ROWS_F = float(M_TOTAL)
EPS = 1e-5


_FPS_SUB = 32
_FPS_LANE = 128


def _fps_body(x_ref, y_ref, z_ref, idx_out_ref, ctr_out_ref):
    X = x_ref[...]
    Y = y_ref[...]
    Z = z_ref[...]
    niota = (jax.lax.broadcasted_iota(jnp.int32, (B, _FPS_SUB, _FPS_LANE), 1) * _FPS_LANE
             + jax.lax.broadcasted_iota(jnp.int32, (B, _FPS_SUB, _FPS_LANE), 2))

    def step(t, carry):
        dist, far = carry
        cmask = niota == far
        cx = jnp.min(jnp.where(cmask, X, 1e9), axis=(1, 2), keepdims=True)
        cy = jnp.min(jnp.where(cmask, Y, 1e9), axis=(1, 2), keepdims=True)
        cz = jnp.min(jnp.where(cmask, Z, 1e9), axis=(1, 2), keepdims=True)
        idx_out_ref[pl.ds(t, 1), :] = far[:, 0, :].reshape(1, B)
        ctr_out_ref[pl.ds(t, 1), :] = jnp.concatenate(
            [cx[:, 0, :], cy[:, 0, :], cz[:, 0, :]], axis=0).reshape(1, 3 * B)
        dx = X - cx
        dy = Y - cy
        dz = Z - cz
        d = (dx * dx + dz * dz) + dy * dy
        dist = jnp.minimum(dist, d)
        m = jnp.max(dist, axis=(1, 2), keepdims=True)
        far = jnp.min(jnp.where(dist == m, niota, jnp.int32(N_POINT)),
                      axis=(1, 2), keepdims=True)
        return dist, far

    init = (jnp.full((B, _FPS_SUB, _FPS_LANE), 1e10, dtype=jnp.float32),
            jnp.zeros((B, 1, 1), dtype=jnp.int32))
    jax.lax.fori_loop(0, N_CENTER, step, init, unroll=False)


def _fps_pallas(xyz):
    xt = jnp.transpose(xyz, (2, 0, 1)).reshape(3, B, _FPS_SUB, _FPS_LANE)
    idx_bs, ctr = pl.pallas_call(
        _fps_body,
        grid=(1,),
        in_specs=[pl.BlockSpec((B, _FPS_SUB, _FPS_LANE), lambda i: (0, 0, 0))] * 3,
        out_specs=[
            pl.BlockSpec((N_CENTER, B), lambda i: (0, 0)),
            pl.BlockSpec((N_CENTER, 3 * B), lambda i: (0, 0)),
        ],
        out_shape=[
            jax.ShapeDtypeStruct((N_CENTER, B), jnp.int32),
            jax.ShapeDtypeStruct((N_CENTER, 3 * B), jnp.float32),
        ],
    )(xt[0], xt[1], xt[2])
    fps_idx = jnp.transpose(idx_bs)
    center_xyz = jnp.transpose(ctr.reshape(N_CENTER, 3, B), (2, 0, 1))
    return fps_idx, center_xyz


def _knn_jax(xyz, k):
    sq = jnp.sum(xyz * xyz, axis=-1)
    dist = sq[:, :, None] + sq[:, None, :] - 2.0 * jnp.einsum('bnc,bmc->bnm', xyz, xyz)
    _, idx = jax.lax.top_k(-dist, k)
    return idx


def _fps_jax(xyz, n_center):
    b, n, _ = xyz.shape

    def step(carry, _):
        dist, far = carry
        centroid = jnp.take_along_axis(xyz, far[:, None, None].astype(jnp.int32), axis=1)
        d = jnp.sum((xyz - centroid) ** 2, axis=-1)
        dist = jnp.minimum(dist, d)
        new_far = jnp.argmax(dist, axis=-1).astype(jnp.int32)
        return (dist, new_far), far

    init = (jnp.full((b, n), 1e10, dtype=jnp.float32), jnp.zeros((b,), dtype=jnp.int32))
    _, idxs = jax.lax.scan(step, init, None, length=n_center)
    return jnp.transpose(idxs)


def _index_points(points, idx):
    return jax.vmap(lambda p, i: p[i])(points, idx)


def _mlp1_body(gfea_ref, gxyz_ref, w1a_ref, w1b_ref, b1_ref,
               h1_ref, stats_ref):
    i = pl.program_id(0)
    h = (jnp.dot(gfea_ref[...], w1a_ref[...], preferred_element_type=jnp.float32)
         + jnp.dot(gxyz_ref[...], w1b_ref[...], preferred_element_type=jnp.float32)
         + b1_ref[...])
    h1_ref[...] = h
    s = jnp.sum(h, axis=0, keepdims=True)
    ss = jnp.sum(h * h, axis=0, keepdims=True)
    upd = jnp.concatenate([s, ss], axis=0)

    @pl.when(i == 0)
    def _():
        stats_ref[...] = upd

    @pl.when(i > 0)
    def _():
        stats_ref[...] += upd


def _mlp_pass1(gfea, gxyz, W1a, W1b, b1):
    grid = (M_TOTAL // BM,)
    return pl.pallas_call(
        _mlp1_body,
        grid=grid,
        in_specs=[
            pl.BlockSpec((BM, C_IN), lambda i: (i, 0)),
            pl.BlockSpec((BM, 8), lambda i: (i, 0)),
            pl.BlockSpec((C_IN, C_MID), lambda i: (0, 0)),
            pl.BlockSpec((8, C_MID), lambda i: (0, 0)),
            pl.BlockSpec((1, C_MID), lambda i: (0, 0)),
        ],
        out_specs=[
            pl.BlockSpec((BM, C_MID), lambda i: (i, 0)),
            pl.BlockSpec((2, C_MID), lambda i: (0, 0)),
        ],
        out_shape=[
            jax.ShapeDtypeStruct((M_TOTAL, C_MID), jnp.float32),
            jax.ShapeDtypeStruct((2, C_MID), jnp.float32),
        ],
    )(gfea, gxyz, W1a, W1b, b1)


def _mlp2_body(h1_ref, stats1_ref, w2_ref, b2_ref, g1_ref, be1_ref,
               maxh2_ref, stats2_ref):
    i = pl.program_id(0)
    m1 = stats1_ref[0:1, :] / N_ROWS_F
    var1 = stats1_ref[1:2, :] / N_ROWS_F - m1 * m1
    inv1 = g1_ref[...] * jax.lax.rsqrt(var1 + EPS)
    a = jnp.maximum((h1_ref[...] - m1) * inv1 + be1_ref[...], 0.0)
    h2 = jnp.dot(a, w2_ref[...], preferred_element_type=jnp.float32) + b2_ref[...]
    s = jnp.sum(h2, axis=0, keepdims=True)
    ss = jnp.sum(h2 * h2, axis=0, keepdims=True)
    upd = jnp.concatenate([s, ss], axis=0)
    maxh2_ref[...] = jnp.max(h2.reshape(BM // N_NEAR, N_NEAR, C_OUT), axis=1)

    @pl.when(i == 0)
    def _():
        stats2_ref[...] = upd

    @pl.when(i > 0)
    def _():
        stats2_ref[...] += upd


def _mlp_pass2(h1, stats1, W2, b2, g1, be1):
    grid = (M_TOTAL // BM,)
    return pl.pallas_call(
        _mlp2_body,
        grid=grid,
        in_specs=[
            pl.BlockSpec((BM, C_MID), lambda i: (i, 0)),
            pl.BlockSpec((2, C_MID), lambda i: (0, 0)),
            pl.BlockSpec((C_MID, C_OUT), lambda i: (0, 0)),
            pl.BlockSpec((1, C_OUT), lambda i: (0, 0)),
            pl.BlockSpec((1, C_MID), lambda i: (0, 0)),
            pl.BlockSpec((1, C_MID), lambda i: (0, 0)),
        ],
        out_specs=[
            pl.BlockSpec((BM // N_NEAR, C_OUT), lambda i: (i, 0)),
            pl.BlockSpec((2, C_OUT), lambda i: (0, 0)),
        ],
        out_shape=[
            jax.ShapeDtypeStruct((B * N_CENTER, C_OUT), jnp.float32),
            jax.ShapeDtypeStruct((2, C_OUT), jnp.float32),
        ],
    )(h1, stats1, W2, b2, g1, be1)


def _epi_body(x_ref, stats2_ref, g2_ref, be2_ref, o_ref):
    m2 = stats2_ref[0:1, :] / N_ROWS_F
    var2 = stats2_ref[1:2, :] / N_ROWS_F - m2 * m2
    inv2 = g2_ref[...] * jax.lax.rsqrt(var2 + EPS)
    o_ref[...] = jnp.maximum((x_ref[...] - m2) * inv2 + be2_ref[...], 0.0)


def _mlp_epilogue(maxh2, stats2, g2, be2):
    grid = (8,)
    bm = (B * N_CENTER) // 8
    return pl.pallas_call(
        _epi_body,
        grid=grid,
        in_specs=[
            pl.BlockSpec((bm, C_OUT), lambda i: (i, 0)),
            pl.BlockSpec((2, C_OUT), lambda i: (0, 0)),
            pl.BlockSpec((1, C_OUT), lambda i: (0, 0)),
            pl.BlockSpec((1, C_OUT), lambda i: (0, 0)),
        ],
        out_specs=pl.BlockSpec((bm, C_OUT), lambda i: (i, 0)),
        out_shape=jax.ShapeDtypeStruct((B * N_CENTER, C_OUT), jnp.float32),
    )(maxh2, stats2, g2, be2)


def kernel(xyz, fea, W1, b1, g1, be1, W2, b2, g2, be2):
    fps_idx, center_xyz = _fps_pallas(xyz)
    sq = jnp.sum(xyz * xyz, axis=-1)
    sqc = jnp.take_along_axis(sq, fps_idx, axis=1)
    dist = (sqc[:, :, None] + sq[:, None, :]
            - 2.0 * jnp.einsum('bsc,bmc->bsm', center_xyz, xyz))
    _, group_idx = jax.lax.top_k(-dist, N_NEAR)
    group_xyz = _index_points(xyz, group_idx)
    xyz_rel = group_xyz - center_xyz[:, :, None, :]
    group_fea = _index_points(fea, group_idx)

    gfea = group_fea.reshape(M_TOTAL, C_IN)
    gxyz = jnp.pad(xyz_rel.reshape(M_TOTAL, 3), ((0, 0), (0, 5)))

    W1a = W1[:C_IN]
    W1b = jnp.pad(W1[C_IN:], ((0, 5), (0, 0)))
    h1, stats1 = _mlp_pass1(gfea, gxyz, W1a, W1b, b1[None, :])
    maxh2, stats2 = _mlp_pass2(h1, stats1, W2, b2[None, :], g1[None, :], be1[None, :])
    out = _mlp_epilogue(maxh2, stats2, g2[None, :], be2[None, :])
    return (center_xyz, out.reshape(B, N_CENTER, C_OUT))


# speedup vs baseline: 3.9621x; 1.3259x over previous
import functools

import jax
import jax.numpy as jnp
import numpy as np
from jax import lax
from jax.experimental import pallas as pl
from jax.experimental.pallas import tpu as pltpu
from jax.experimental.pallas import tpu_sc as plsc

B = 4
N_POINT = 4096
N_CENTER = 1024
N_NEAR = 32
C_IN = 128
C_MID = 183
C_OUT = 256

BM = 512
M_TOTAL = B * N_CENTER * N_NEAR
N_ROWS_F = float(M_TOTAL)
EPS = 1e-5


_FPS_SUB = 32
_FPS_LANE = 128


def _fps_body(x_ref, y_ref, z_ref, idx_out_ref, ctr_out_ref):
    X = x_ref[...]
    Y = y_ref[...]
    Z = z_ref[...]
    niota = (jax.lax.broadcasted_iota(jnp.int32, (B, _FPS_SUB, _FPS_LANE), 1) * _FPS_LANE
             + jax.lax.broadcasted_iota(jnp.int32, (B, _FPS_SUB, _FPS_LANE), 2))

    def step(t, carry):
        dist, far = carry
        cmask = niota == far
        cx = jnp.min(jnp.where(cmask, X, 1e9), axis=(1, 2), keepdims=True)
        cy = jnp.min(jnp.where(cmask, Y, 1e9), axis=(1, 2), keepdims=True)
        cz = jnp.min(jnp.where(cmask, Z, 1e9), axis=(1, 2), keepdims=True)
        idx_out_ref[pl.ds(t, 1), :] = far[:, 0, :].reshape(1, B)
        ctr_out_ref[pl.ds(t, 1), :] = jnp.concatenate(
            [cx[:, 0, :], cy[:, 0, :], cz[:, 0, :]], axis=0).reshape(1, 3 * B)
        dx = X - cx
        dy = Y - cy
        dz = Z - cz
        d = (dx * dx + dz * dz) + dy * dy
        dist = jnp.minimum(dist, d)
        m = jnp.max(dist, axis=(1, 2), keepdims=True)
        far = jnp.min(jnp.where(dist == m, niota, jnp.int32(N_POINT)),
                      axis=(1, 2), keepdims=True)
        return dist, far

    init = (jnp.full((B, _FPS_SUB, _FPS_LANE), 1e10, dtype=jnp.float32),
            jnp.zeros((B, 1, 1), dtype=jnp.int32))
    jax.lax.fori_loop(0, N_CENTER, step, init, unroll=False)


def _fps_pallas(xyz):
    xt = jnp.transpose(xyz, (2, 0, 1)).reshape(3, B, _FPS_SUB, _FPS_LANE)
    idx_bs, ctr = pl.pallas_call(
        _fps_body,
        grid=(1,),
        in_specs=[pl.BlockSpec((B, _FPS_SUB, _FPS_LANE), lambda i: (0, 0, 0))] * 3,
        out_specs=[
            pl.BlockSpec((N_CENTER, B), lambda i: (0, 0)),
            pl.BlockSpec((N_CENTER, 3 * B), lambda i: (0, 0)),
        ],
        out_shape=[
            jax.ShapeDtypeStruct((N_CENTER, B), jnp.int32),
            jax.ShapeDtypeStruct((N_CENTER, 3 * B), jnp.float32),
        ],
    )(xt[0], xt[1], xt[2])
    fps_idx = jnp.transpose(idx_bs)
    center_xyz = jnp.transpose(ctr.reshape(N_CENTER, 3, B), (2, 0, 1))
    return fps_idx, center_xyz


_GW = 32
_G_PER_W = M_TOTAL // _GW
_GCHUNK = 512
_GN_CHUNKS = _G_PER_W // _GCHUNK


def _sc_gather(gidx, fea2):
    mesh = plsc.VectorSubcoreMesh(core_axis_name="c", subcore_axis_name="s")

    @functools.partial(
        pl.kernel,
        mesh=mesh,
        out_type=jax.ShapeDtypeStruct((M_TOTAL, C_IN), jnp.float32),
        scratch_types=[
            pltpu.VMEM((_GCHUNK,), jnp.int32),
            pltpu.VMEM((_GCHUNK, C_IN), jnp.float32),
            pltpu.SemaphoreType.DMA,
        ],
    )
    def body(idx_hbm, fea_hbm, out_fea, idx_v, fea_v, s1):
        wid = lax.axis_index("s") * 2 + lax.axis_index("c")

        def chunk(c, carry):
            base = wid * _G_PER_W + c * _GCHUNK
            pltpu.sync_copy(idx_hbm.at[pl.ds(base, _GCHUNK)], idx_v)
            pltpu.async_copy(fea_hbm.at[idx_v], fea_v, s1).wait()
            pltpu.sync_copy(fea_v, out_fea.at[pl.ds(base, _GCHUNK)])
            return carry

        lax.fori_loop(0, _GN_CHUNKS, chunk, 0)

    return body(gidx, fea2)


def _knn_jax(xyz, k):
    sq = jnp.sum(xyz * xyz, axis=-1)
    dist = sq[:, :, None] + sq[:, None, :] - 2.0 * jnp.einsum('bnc,bmc->bnm', xyz, xyz)
    _, idx = jax.lax.top_k(-dist, k)
    return idx


def _fps_jax(xyz, n_center):
    b, n, _ = xyz.shape

    def step(carry, _):
        dist, far = carry
        centroid = jnp.take_along_axis(xyz, far[:, None, None].astype(jnp.int32), axis=1)
        d = jnp.sum((xyz - centroid) ** 2, axis=-1)
        dist = jnp.minimum(dist, d)
        new_far = jnp.argmax(dist, axis=-1).astype(jnp.int32)
        return (dist, new_far), far

    init = (jnp.full((b, n), 1e10, dtype=jnp.float32), jnp.zeros((b,), dtype=jnp.int32))
    _, idxs = jax.lax.scan(step, init, None, length=n_center)
    return jnp.transpose(idxs)


def _index_points(points, idx):
    return jax.vmap(lambda p, i: p[i])(points, idx)


def _mlp1_body(gfea_ref, gxyz_ref, w1a_ref, w1b_ref, b1_ref,
               h1_ref, stats_ref):
    i = pl.program_id(0)
    h = (jnp.dot(gfea_ref[...], w1a_ref[...], preferred_element_type=jnp.float32)
         + jnp.dot(gxyz_ref[...], w1b_ref[...], preferred_element_type=jnp.float32)
         + b1_ref[...])
    h1_ref[...] = h
    s = jnp.sum(h, axis=0, keepdims=True)
    ss = jnp.sum(h * h, axis=0, keepdims=True)
    upd = jnp.concatenate([s, ss], axis=0)

    @pl.when(i == 0)
    def _():
        stats_ref[...] = upd

    @pl.when(i > 0)
    def _():
        stats_ref[...] += upd


def _mlp_pass1(gfea, gxyz, W1a, W1b, b1):
    grid = (M_TOTAL // BM,)
    return pl.pallas_call(
        _mlp1_body,
        grid=grid,
        in_specs=[
            pl.BlockSpec((BM, C_IN), lambda i: (i, 0)),
            pl.BlockSpec((BM, 16), lambda i: (i, 0)),
            pl.BlockSpec((C_IN, C_MID), lambda i: (0, 0)),
            pl.BlockSpec((16, C_MID), lambda i: (0, 0)),
            pl.BlockSpec((1, C_MID), lambda i: (0, 0)),
        ],
        out_specs=[
            pl.BlockSpec((BM, C_MID), lambda i: (i, 0)),
            pl.BlockSpec((2, C_MID), lambda i: (0, 0)),
        ],
        out_shape=[
            jax.ShapeDtypeStruct((M_TOTAL, C_MID), jnp.float32),
            jax.ShapeDtypeStruct((2, C_MID), jnp.float32),
        ],
    )(gfea, gxyz, W1a, W1b, b1)


def _mlp2_body(h1_ref, stats1_ref, w2_ref, b2_ref, g1_ref, be1_ref,
               maxh2_ref, stats2_ref):
    i = pl.program_id(0)
    m1 = stats1_ref[0:1, :] / N_ROWS_F
    var1 = stats1_ref[1:2, :] / N_ROWS_F - m1 * m1
    inv1 = g1_ref[...] * jax.lax.rsqrt(var1 + EPS)
    a = jnp.maximum((h1_ref[...] - m1) * inv1 + be1_ref[...], 0.0)
    h2 = jnp.dot(a, w2_ref[...], preferred_element_type=jnp.float32) + b2_ref[...]
    s = jnp.sum(h2, axis=0, keepdims=True)
    ss = jnp.sum(h2 * h2, axis=0, keepdims=True)
    upd = jnp.concatenate([s, ss], axis=0)
    maxh2_ref[...] = jnp.max(h2.reshape(BM // N_NEAR, N_NEAR, C_OUT), axis=1)

    @pl.when(i == 0)
    def _():
        stats2_ref[...] = upd

    @pl.when(i > 0)
    def _():
        stats2_ref[...] += upd


def _mlp_pass2(h1, stats1, W2, b2, g1, be1):
    grid = (M_TOTAL // BM,)
    return pl.pallas_call(
        _mlp2_body,
        grid=grid,
        in_specs=[
            pl.BlockSpec((BM, C_MID), lambda i: (i, 0)),
            pl.BlockSpec((2, C_MID), lambda i: (0, 0)),
            pl.BlockSpec((C_MID, C_OUT), lambda i: (0, 0)),
            pl.BlockSpec((1, C_OUT), lambda i: (0, 0)),
            pl.BlockSpec((1, C_MID), lambda i: (0, 0)),
            pl.BlockSpec((1, C_MID), lambda i: (0, 0)),
        ],
        out_specs=[
            pl.BlockSpec((BM // N_NEAR, C_OUT), lambda i: (i, 0)),
            pl.BlockSpec((2, C_OUT), lambda i: (0, 0)),
        ],
        out_shape=[
            jax.ShapeDtypeStruct((B * N_CENTER, C_OUT), jnp.float32),
            jax.ShapeDtypeStruct((2, C_OUT), jnp.float32),
        ],
    )(h1, stats1, W2, b2, g1, be1)


def _epi_body(x_ref, stats2_ref, g2_ref, be2_ref, o_ref):
    m2 = stats2_ref[0:1, :] / N_ROWS_F
    var2 = stats2_ref[1:2, :] / N_ROWS_F - m2 * m2
    inv2 = g2_ref[...] * jax.lax.rsqrt(var2 + EPS)
    o_ref[...] = jnp.maximum((x_ref[...] - m2) * inv2 + be2_ref[...], 0.0)


def _mlp_epilogue(maxh2, stats2, g2, be2):
    grid = (8,)
    bm = (B * N_CENTER) // 8
    return pl.pallas_call(
        _epi_body,
        grid=grid,
        in_specs=[
            pl.BlockSpec((bm, C_OUT), lambda i: (i, 0)),
            pl.BlockSpec((2, C_OUT), lambda i: (0, 0)),
            pl.BlockSpec((1, C_OUT), lambda i: (0, 0)),
            pl.BlockSpec((1, C_OUT), lambda i: (0, 0)),
        ],
        out_specs=pl.BlockSpec((bm, C_OUT), lambda i: (i, 0)),
        out_shape=jax.ShapeDtypeStruct((B * N_CENTER, C_OUT), jnp.float32),
    )(maxh2, stats2, g2, be2)


def kernel(xyz, fea, W1, b1, g1, be1, W2, b2, g2, be2):
    fps_idx, center_xyz = _fps_pallas(xyz)
    sq = jnp.sum(xyz * xyz, axis=-1)
    sqc = jnp.take_along_axis(sq, fps_idx, axis=1)
    dist = (sqc[:, :, None] + sq[:, None, :]
            - 2.0 * jnp.einsum('bsc,bmc->bsm', center_xyz, xyz))
    _, group_idx = jax.lax.top_k(-dist, N_NEAR)

    gidx = (group_idx.astype(jnp.int32)
            + (jnp.arange(B, dtype=jnp.int32) * N_POINT)[:, None, None]
            ).reshape(M_TOTAL)
    fea2 = fea.reshape(B * N_POINT, C_IN)
    gfea = _sc_gather(gidx, fea2)

    group_xyz = _index_points(xyz, group_idx)
    xyz_rel = group_xyz - center_xyz[:, :, None, :]
    gxyz = jnp.pad(xyz_rel.reshape(M_TOTAL, 3), ((0, 0), (0, 13)))

    W1a = W1[:C_IN]
    W1b = jnp.pad(W1[C_IN:], ((0, 13), (0, 0)))
    h1, stats1 = _mlp_pass1(gfea, gxyz, W1a, W1b, b1[None, :])
    maxh2, stats2 = _mlp_pass2(h1, stats1, W2, b2[None, :], g1[None, :], be1[None, :])
    out = _mlp_epilogue(maxh2, stats2, g2[None, :], be2[None, :])
    return (center_xyz, out.reshape(B, N_CENTER, C_OUT))


# speedup vs baseline: 5.8486x; 1.4761x over previous
import functools

import jax
import jax.numpy as jnp
import numpy as np
from jax import lax
from jax.experimental import pallas as pl
from jax.experimental.pallas import tpu as pltpu
from jax.experimental.pallas import tpu_sc as plsc

B = 4
N_POINT = 4096
N_CENTER = 1024
N_NEAR = 32
C_IN = 128
C_MID = 183
C_OUT = 256

BM = 512
M_TOTAL = B * N_CENTER * N_NEAR
N_ROWS_F = float(M_TOTAL)
EPS = 1e-5


_FPS_SUB = 32
_FPS_LANE = 128


def _fps_body(x_ref, y_ref, z_ref, idx_out_ref, ctr_out_ref):
    X = x_ref[...]
    Y = y_ref[...]
    Z = z_ref[...]
    niota = (jax.lax.broadcasted_iota(jnp.int32, (B, _FPS_SUB, _FPS_LANE), 1) * _FPS_LANE
             + jax.lax.broadcasted_iota(jnp.int32, (B, _FPS_SUB, _FPS_LANE), 2))

    def step(t, carry):
        dist, far = carry
        cmask = niota == far
        cx = jnp.min(jnp.where(cmask, X, 1e9), axis=(1, 2), keepdims=True)
        cy = jnp.min(jnp.where(cmask, Y, 1e9), axis=(1, 2), keepdims=True)
        cz = jnp.min(jnp.where(cmask, Z, 1e9), axis=(1, 2), keepdims=True)
        idx_out_ref[pl.ds(t, 1), :] = far[:, 0, :].reshape(1, B)
        ctr_out_ref[pl.ds(t, 1), :] = jnp.concatenate(
            [cx[:, 0, :], cy[:, 0, :], cz[:, 0, :]], axis=0).reshape(1, 3 * B)
        dx = X - cx
        dy = Y - cy
        dz = Z - cz
        d = (dx * dx + dz * dz) + dy * dy
        dist = jnp.minimum(dist, d)
        m = jnp.max(dist, axis=(1, 2), keepdims=True)
        far = jnp.min(jnp.where(dist == m, niota, jnp.int32(N_POINT)),
                      axis=(1, 2), keepdims=True)
        return dist, far

    init = (jnp.full((B, _FPS_SUB, _FPS_LANE), 1e10, dtype=jnp.float32),
            jnp.zeros((B, 1, 1), dtype=jnp.int32))
    jax.lax.fori_loop(0, N_CENTER, step, init, unroll=False)


def _fps_pallas(xyz):
    xt = jnp.transpose(xyz, (2, 0, 1)).reshape(3, B, _FPS_SUB, _FPS_LANE)
    idx_bs, ctr = pl.pallas_call(
        _fps_body,
        grid=(1,),
        in_specs=[pl.BlockSpec((B, _FPS_SUB, _FPS_LANE), lambda i: (0, 0, 0))] * 3,
        out_specs=[
            pl.BlockSpec((N_CENTER, B), lambda i: (0, 0)),
            pl.BlockSpec((N_CENTER, 3 * B), lambda i: (0, 0)),
        ],
        out_shape=[
            jax.ShapeDtypeStruct((N_CENTER, B), jnp.int32),
            jax.ShapeDtypeStruct((N_CENTER, 3 * B), jnp.float32),
        ],
    )(xt[0], xt[1], xt[2])
    fps_idx = jnp.transpose(idx_bs)
    center_xyz = jnp.transpose(ctr.reshape(N_CENTER, 3, B), (2, 0, 1))
    return fps_idx, center_xyz


_GW = 32
_G_PER_W = M_TOTAL // _GW
_GCHUNK = 256
_GN_CHUNKS = _G_PER_W // _GCHUNK
_GD = C_IN + 128


def _sc_gather(gidx, table):
    mesh = plsc.VectorSubcoreMesh(core_axis_name="c", subcore_axis_name="s")

    @functools.partial(
        pl.kernel,
        mesh=mesh,
        out_type=jax.ShapeDtypeStruct((M_TOTAL, _GD), jnp.float32),
        scratch_types=[
            pltpu.VMEM((_GCHUNK,), jnp.int32),
            pltpu.VMEM((_GCHUNK, _GD), jnp.float32),
            pltpu.SemaphoreType.DMA,
        ],
    )
    def body(idx_hbm, tab_hbm, out_hbm, idx_v, rows_v, s1):
        wid = lax.axis_index("s") * 2 + lax.axis_index("c")

        def chunk(c, carry):
            base = wid * _G_PER_W + c * _GCHUNK
            pltpu.sync_copy(idx_hbm.at[pl.ds(base, _GCHUNK)], idx_v)
            pltpu.async_copy(tab_hbm.at[idx_v], rows_v, s1).wait()
            pltpu.sync_copy(rows_v, out_hbm.at[pl.ds(base, _GCHUNK)])
            return carry

        lax.fori_loop(0, _GN_CHUNKS, chunk, 0)

    return body(gidx, table)


def _knn_jax(xyz, k):
    sq = jnp.sum(xyz * xyz, axis=-1)
    dist = sq[:, :, None] + sq[:, None, :] - 2.0 * jnp.einsum('bnc,bmc->bnm', xyz, xyz)
    _, idx = jax.lax.top_k(-dist, k)
    return idx


def _fps_jax(xyz, n_center):
    b, n, _ = xyz.shape

    def step(carry, _):
        dist, far = carry
        centroid = jnp.take_along_axis(xyz, far[:, None, None].astype(jnp.int32), axis=1)
        d = jnp.sum((xyz - centroid) ** 2, axis=-1)
        dist = jnp.minimum(dist, d)
        new_far = jnp.argmax(dist, axis=-1).astype(jnp.int32)
        return (dist, new_far), far

    init = (jnp.full((b, n), 1e10, dtype=jnp.float32), jnp.zeros((b,), dtype=jnp.int32))
    _, idxs = jax.lax.scan(step, init, None, length=n_center)
    return jnp.transpose(idxs)


def _index_points(points, idx):
    return jax.vmap(lambda p, i: p[i])(points, idx)


def _mlp1_body(g_ref, ctr_ref, w1a_ref, w1b_ref, b1_ref,
               h1_ref, stats_ref):
    i = pl.program_id(0)
    gfea = g_ref[:, :C_IN]
    gx = (g_ref[:, C_IN:C_IN + 16].reshape(BM // N_NEAR, N_NEAR, 16)
          - ctr_ref[...][:, None, :]).reshape(BM, 16)
    h = (jnp.dot(gfea, w1a_ref[...], preferred_element_type=jnp.float32)
         + jnp.dot(gx, w1b_ref[...], preferred_element_type=jnp.float32)
         + b1_ref[...])
    h1_ref[...] = h
    s = jnp.sum(h, axis=0, keepdims=True)
    ss = jnp.sum(h * h, axis=0, keepdims=True)
    upd = jnp.concatenate([s, ss], axis=0)

    @pl.when(i == 0)
    def _():
        stats_ref[...] = upd

    @pl.when(i > 0)
    def _():
        stats_ref[...] += upd


def _mlp_pass1(gcomb, ctr16, W1a, W1b, b1):
    grid = (M_TOTAL // BM,)
    return pl.pallas_call(
        _mlp1_body,
        grid=grid,
        in_specs=[
            pl.BlockSpec((BM, _GD), lambda i: (i, 0)),
            pl.BlockSpec((BM // N_NEAR, 16), lambda i: (i, 0)),
            pl.BlockSpec((C_IN, C_MID), lambda i: (0, 0)),
            pl.BlockSpec((16, C_MID), lambda i: (0, 0)),
            pl.BlockSpec((1, C_MID), lambda i: (0, 0)),
        ],
        out_specs=[
            pl.BlockSpec((BM, C_MID), lambda i: (i, 0)),
            pl.BlockSpec((2, C_MID), lambda i: (0, 0)),
        ],
        out_shape=[
            jax.ShapeDtypeStruct((M_TOTAL, C_MID), jnp.float32),
            jax.ShapeDtypeStruct((2, C_MID), jnp.float32),
        ],
    )(gcomb, ctr16, W1a, W1b, b1)


def _mlp2_body(h1_ref, stats1_ref, w2_ref, b2_ref, g1_ref, be1_ref,
               maxh2_ref, stats2_ref):
    i = pl.program_id(0)
    m1 = stats1_ref[0:1, :] / N_ROWS_F
    var1 = stats1_ref[1:2, :] / N_ROWS_F - m1 * m1
    inv1 = g1_ref[...] * jax.lax.rsqrt(var1 + EPS)
    a = jnp.maximum((h1_ref[...] - m1) * inv1 + be1_ref[...], 0.0)
    h2 = jnp.dot(a, w2_ref[...], preferred_element_type=jnp.float32) + b2_ref[...]
    s = jnp.sum(h2, axis=0, keepdims=True)
    ss = jnp.sum(h2 * h2, axis=0, keepdims=True)
    upd = jnp.concatenate([s, ss], axis=0)
    maxh2_ref[...] = jnp.max(h2.reshape(BM // N_NEAR, N_NEAR, C_OUT), axis=1)

    @pl.when(i == 0)
    def _():
        stats2_ref[...] = upd

    @pl.when(i > 0)
    def _():
        stats2_ref[...] += upd


def _mlp_pass2(h1, stats1, W2, b2, g1, be1):
    grid = (M_TOTAL // BM,)
    return pl.pallas_call(
        _mlp2_body,
        grid=grid,
        in_specs=[
            pl.BlockSpec((BM, C_MID), lambda i: (i, 0)),
            pl.BlockSpec((2, C_MID), lambda i: (0, 0)),
            pl.BlockSpec((C_MID, C_OUT), lambda i: (0, 0)),
            pl.BlockSpec((1, C_OUT), lambda i: (0, 0)),
            pl.BlockSpec((1, C_MID), lambda i: (0, 0)),
            pl.BlockSpec((1, C_MID), lambda i: (0, 0)),
        ],
        out_specs=[
            pl.BlockSpec((BM // N_NEAR, C_OUT), lambda i: (i, 0)),
            pl.BlockSpec((2, C_OUT), lambda i: (0, 0)),
        ],
        out_shape=[
            jax.ShapeDtypeStruct((B * N_CENTER, C_OUT), jnp.float32),
            jax.ShapeDtypeStruct((2, C_OUT), jnp.float32),
        ],
    )(h1, stats1, W2, b2, g1, be1)


def _epi_body(x_ref, stats2_ref, g2_ref, be2_ref, o_ref):
    m2 = stats2_ref[0:1, :] / N_ROWS_F
    var2 = stats2_ref[1:2, :] / N_ROWS_F - m2 * m2
    inv2 = g2_ref[...] * jax.lax.rsqrt(var2 + EPS)
    o_ref[...] = jnp.maximum((x_ref[...] - m2) * inv2 + be2_ref[...], 0.0)


def _mlp_epilogue(maxh2, stats2, g2, be2):
    grid = (8,)
    bm = (B * N_CENTER) // 8
    return pl.pallas_call(
        _epi_body,
        grid=grid,
        in_specs=[
            pl.BlockSpec((bm, C_OUT), lambda i: (i, 0)),
            pl.BlockSpec((2, C_OUT), lambda i: (0, 0)),
            pl.BlockSpec((1, C_OUT), lambda i: (0, 0)),
            pl.BlockSpec((1, C_OUT), lambda i: (0, 0)),
        ],
        out_specs=pl.BlockSpec((bm, C_OUT), lambda i: (i, 0)),
        out_shape=jax.ShapeDtypeStruct((B * N_CENTER, C_OUT), jnp.float32),
    )(maxh2, stats2, g2, be2)


def kernel(xyz, fea, W1, b1, g1, be1, W2, b2, g2, be2):
    fps_idx, center_xyz = _fps_pallas(xyz)
    sq = jnp.sum(xyz * xyz, axis=-1)
    sqc = jnp.take_along_axis(sq, fps_idx, axis=1)
    dist = (sqc[:, :, None] + sq[:, None, :]
            - 2.0 * jnp.einsum('bsc,bmc->bsm', center_xyz, xyz))
    _, group_idx = jax.lax.top_k(-dist, N_NEAR)

    gidx = (group_idx.astype(jnp.int32)
            + (jnp.arange(B, dtype=jnp.int32) * N_POINT)[:, None, None]
            ).reshape(M_TOTAL)
    fea2 = fea.reshape(B * N_POINT, C_IN)
    xyzp = jnp.pad(xyz.reshape(B * N_POINT, 3), ((0, 0), (0, 125)))
    table = jnp.concatenate([fea2, xyzp], axis=1)
    gcomb = _sc_gather(gidx, table)

    ctr16 = jnp.pad(center_xyz.reshape(B * N_CENTER, 3), ((0, 0), (0, 13)))
    W1a = W1[:C_IN]
    W1b = jnp.pad(W1[C_IN:], ((0, 13), (0, 0)))
    h1, stats1 = _mlp_pass1(gcomb, ctr16, W1a, W1b, b1[None, :])
    maxh2, stats2 = _mlp_pass2(h1, stats1, W2, b2[None, :], g1[None, :], be1[None, :])
    out = _mlp_epilogue(maxh2, stats2, g2[None, :], be2[None, :])
    return (center_xyz, out.reshape(B, N_CENTER, C_OUT))
